# baseline (speedup 1.0000x reference)
import functools

import jax
import jax.numpy as jnp
from jax import lax
from jax.experimental import pallas as pl
from jax.experimental.pallas import tpu as pltpu
from jax.experimental.pallas import tpu_sc as plsc

_B = 1024
_L = 200
_EMB = 128
_CHUNK = 128
_CAT_ROWS = _B * 2 * _L
_BE_ROWS = _B * _L
_CAT_CHUNKS = _CAT_ROWS // _CHUNK
_BE_CHUNKS = _BE_ROWS // _CHUNK
_NC = 2
_NS = 16
_NW = _NC * _NS
_CAT_PER_W = _CAT_CHUNKS // _NW
_BE_PER_W = _BE_CHUNKS // _NW
_TAB = 256
_OFF_PITCH = 64
_OFF_DUR = 192


@functools.lru_cache(maxsize=1)
def _make_lookup():
  mesh = plsc.VectorSubcoreMesh(
      core_axis_name="c", subcore_axis_name="s", num_cores=_NC)

  @functools.partial(
      pl.kernel,
      mesh=mesh,
      out_type=[
          jax.ShapeDtypeStruct((_CAT_ROWS, _EMB), jnp.float32),
          jax.ShapeDtypeStruct((_BE_ROWS, _EMB), jnp.float32),
      ],
      scratch_types=[
          pltpu.VMEM((_CAT_PER_W, _CHUNK), jnp.int32),
          pltpu.VMEM((_BE_PER_W, _CHUNK), jnp.int32),
          pltpu.VMEM((2 * _CHUNK, _EMB), jnp.float32),
          pltpu.VMEM((2 * _CHUNK, _EMB), jnp.float32),
          pltpu.VMEM_SHARED((_TAB, _EMB), jnp.float32),
          pltpu.SemaphoreType.DMA,
          pltpu.SemaphoreType.DMA,
      ],
  )
  def lookup(pitch_hbm, dur_hbm, beat_hbm, idxcat_hbm, idxbe_hbm,
             outcat_hbm, outbe_hbm,
             idxc_v, idxb_v, buf0, buf1, table_sp, sem0, sem1):
    sid = lax.axis_index("s")
    wid = sid * _NC + lax.axis_index("c")

    @pl.when(sid == 0)
    def _():
      pltpu.sync_copy(beat_hbm, table_sp.at[pl.ds(0, _OFF_PITCH)])
      pltpu.sync_copy(pitch_hbm, table_sp.at[pl.ds(_OFF_PITCH, 128)])
      pltpu.sync_copy(dur_hbm, table_sp.at[pl.ds(_OFF_DUR, 64)])

    pltpu.async_copy(idxcat_hbm.at[wid], idxc_v, sem0)
    pltpu.async_copy(idxbe_hbm.at[wid], idxb_v, sem1)
    pltpu.make_async_copy(idxcat_hbm.at[0], idxc_v, sem0).wait()
    pltpu.make_async_copy(idxbe_hbm.at[0], idxb_v, sem1).wait()
    plsc.subcore_barrier()

    def fire(idx_v, i, buf, sem):
      pltpu.async_copy(table_sp.at[idx_v.at[2 * i]],
                       buf.at[pl.ds(0, _CHUNK)], sem)
      pltpu.async_copy(table_sp.at[idx_v.at[2 * i + 1]],
                       buf.at[pl.ds(_CHUNK, _CHUNK)], sem)

    def drain(idx_v, buf, sem):
      pltpu.make_async_copy(table_sp.at[idx_v.at[0]],
                            buf.at[pl.ds(0, _CHUNK)], sem).wait()
      pltpu.make_async_copy(table_sp.at[idx_v.at[0]],
                            buf.at[pl.ds(_CHUNK, _CHUNK)], sem).wait()

    def step(idx_v, out_hbm, base, i, nbig, buf, sem):
      drain(idx_v, buf, sem)
      pltpu.sync_copy(buf,
                      out_hbm.at[pl.ds((base + i) * 2 * _CHUNK, 2 * _CHUNK)])

      @pl.when(i + 2 < nbig)
      def _():
        fire(idx_v, i + 2, buf, sem)

    def field(idx_v, out_hbm, per_w):
      nbig = per_w // 2
      base = wid * nbig
      fire(idx_v, 0, buf0, sem0)

      @pl.when(nbig > 1)
      def _():
        fire(idx_v, 1, buf1, sem1)

      def body(i, carry):
        @pl.when(i % 2 == 0)
        def _():
          step(idx_v, out_hbm, base, i, nbig, buf0, sem0)

        @pl.when(i % 2 == 1)
        def _():
          step(idx_v, out_hbm, base, i, nbig, buf1, sem1)

        return carry

      lax.fori_loop(0, nbig, body, 0)

    field(idxc_v, outcat_hbm, _CAT_PER_W)
    field(idxb_v, outbe_hbm, _BE_PER_W)

  return lookup


def kernel(x, beat_info, pitch_emb, beat_emb, dur_emb):
  pitch = x[..., 2]
  dur = x[..., 3]
  idx_cat = jnp.concatenate([pitch + _OFF_PITCH, dur + _OFF_DUR],
                            axis=1).reshape(_NW, _CAT_PER_W, _CHUNK)
  idx_be = beat_info.reshape(_NW, _BE_PER_W, _CHUNK)
  out_cat_flat, be_flat = _make_lookup()(
      pitch_emb, dur_emb, beat_emb, idx_cat, idx_be)
  out_cat = out_cat_flat.reshape(_B, 2 * _L, _EMB)
  be = be_flat.reshape(_B, _L, _EMB)
  return (out_cat, be, beat_info, pitch, dur)


# speedup vs baseline: 1.0078x; 1.0078x over previous
import functools

import jax
import jax.numpy as jnp
from jax import lax
from jax.experimental import pallas as pl
from jax.experimental.pallas import tpu as pltpu
from jax.experimental.pallas import tpu_sc as plsc

_B = 1024
_L = 200
_EMB = 128
_CHUNK = 128
_CAT_ROWS = _B * 2 * _L
_BE_ROWS = _B * _L
_CAT_CHUNKS = _CAT_ROWS // _CHUNK
_BE_CHUNKS = _BE_ROWS // _CHUNK
_NC = 2
_NS = 16
_NW = _NC * _NS
_CAT_PER_W = _CAT_CHUNKS // _NW
_BE_PER_W = _BE_CHUNKS // _NW
_TAB = 256
_OFF_PITCH = 64
_OFF_DUR = 192


@functools.lru_cache(maxsize=1)
def _make_lookup():
  mesh = plsc.VectorSubcoreMesh(
      core_axis_name="c", subcore_axis_name="s", num_cores=_NC)

  @functools.partial(
      pl.kernel,
      mesh=mesh,
      out_type=[
          jax.ShapeDtypeStruct((_CAT_ROWS, _EMB), jnp.float32),
          jax.ShapeDtypeStruct((_BE_ROWS, _EMB), jnp.float32),
      ],
      scratch_types=[
          pltpu.VMEM((_CAT_PER_W, _CHUNK), jnp.int32),
          pltpu.VMEM((_BE_PER_W, _CHUNK), jnp.int32),
          pltpu.VMEM((_CHUNK, _EMB), jnp.float32),
          pltpu.VMEM((_CHUNK, _EMB), jnp.float32),
          pltpu.VMEM_SHARED((_TAB, _EMB), jnp.float32),
          pltpu.SemaphoreType.DMA,
          pltpu.SemaphoreType.DMA,
      ],
  )
  def lookup(pitch_hbm, dur_hbm, beat_hbm, idxcat_hbm, idxbe_hbm,
             outcat_hbm, outbe_hbm,
             idxc_v, idxb_v, buf0, buf1, table_sp, sem0, sem1):
    sid = lax.axis_index("s")
    wid = sid * _NC + lax.axis_index("c")

    @pl.when(sid == 0)
    def _():
      pltpu.sync_copy(beat_hbm, table_sp.at[pl.ds(0, _OFF_PITCH)])
      pltpu.sync_copy(pitch_hbm, table_sp.at[pl.ds(_OFF_PITCH, 128)])
      pltpu.sync_copy(dur_hbm, table_sp.at[pl.ds(_OFF_DUR, 64)])

    pltpu.async_copy(idxcat_hbm.at[wid], idxc_v, sem0)
    pltpu.async_copy(idxbe_hbm.at[wid], idxb_v, sem1)
    pltpu.make_async_copy(idxcat_hbm.at[0], idxc_v, sem0).wait()
    pltpu.make_async_copy(idxbe_hbm.at[0], idxb_v, sem1).wait()
    plsc.subcore_barrier()

    def fire(idx_v, j, buf, sem):
      pltpu.async_copy(table_sp.at[idx_v.at[j]], buf, sem)

    def drain(idx_v, buf, sem):
      pltpu.make_async_copy(table_sp.at[idx_v.at[0]], buf, sem).wait()

    def field(idx_v, out_hbm, per_w):
      base = wid * per_w
      fire(idx_v, 0, buf0, sem0)
      fire(idx_v, 1, buf1, sem1)

      def body(k, carry):
        i0 = 2 * k
        i1 = i0 + 1
        drain(idx_v, buf0, sem0)
        pltpu.sync_copy(buf0, out_hbm.at[pl.ds((base + i0) * _CHUNK, _CHUNK)])

        @pl.when(i0 + 2 < per_w)
        def _():
          fire(idx_v, i0 + 2, buf0, sem0)

        drain(idx_v, buf1, sem1)
        pltpu.sync_copy(buf1, out_hbm.at[pl.ds((base + i1) * _CHUNK, _CHUNK)])

        @pl.when(i1 + 2 < per_w)
        def _():
          fire(idx_v, i1 + 2, buf1, sem1)

        return carry

      lax.fori_loop(0, per_w // 2, body, 0)

    field(idxc_v, outcat_hbm, _CAT_PER_W)
    field(idxb_v, outbe_hbm, _BE_PER_W)

  return lookup


def kernel(x, beat_info, pitch_emb, beat_emb, dur_emb):
  pitch = x[..., 2]
  dur = x[..., 3]
  idx_cat = jnp.concatenate([pitch + _OFF_PITCH, dur + _OFF_DUR],
                            axis=1).reshape(_NW, _CAT_PER_W, _CHUNK)
  idx_be = beat_info.reshape(_NW, _BE_PER_W, _CHUNK)
  out_cat_flat, be_flat = _make_lookup()(
      pitch_emb, dur_emb, beat_emb, idx_cat, idx_be)
  out_cat = out_cat_flat.reshape(_B, 2 * _L, _EMB)
  be = be_flat.reshape(_B, _L, _EMB)
  return (out_cat, be, beat_info, pitch, dur)
